# baseline (speedup 1.0000x reference)
import functools

import jax
import jax.numpy as jnp
from jax import lax
from jax.experimental import pallas as pl
from jax.experimental.pallas import tpu as pltpu
from jax.experimental.pallas import tpu_sc as plsc

RADIUS = 0.2
NS = 32
B, N, C = 4, 4096, 64
BN = B * N
DP = 80
NW = 32
PPW = BN // NW
GP = 4
NCH = N // 16
UNR = 16


def _sc_ball_gather(xs, ys, zs, gtab):
    mesh = plsc.VectorSubcoreMesh(core_axis_name="c", subcore_axis_name="s")

    @functools.partial(
        pl.kernel,
        out_type=jax.ShapeDtypeStruct((BN * NS, DP), jnp.float32),
        mesh=mesh,
        scratch_types=[
            pltpu.VMEM((N,), jnp.float32),
            pltpu.VMEM((N,), jnp.float32),
            pltpu.VMEM((N,), jnp.float32),
            pltpu.VMEM((320,), jnp.int32),
            pltpu.VMEM((GP * NS,), jnp.int32),
            pltpu.VMEM((GP * NS,), jnp.int32),
            pltpu.VMEM((GP * NS, DP), jnp.float32),
            pltpu.VMEM((GP * NS, DP), jnp.float32),
            pltpu.SemaphoreType.DMA,
            pltpu.SemaphoreType.DMA,
            pltpu.SemaphoreType.DMA,
            pltpu.SemaphoreType.DMA,
        ],
        compiler_params=pltpu.CompilerParams(
            needs_layout_passes=False, use_tc_tiling_on_sc=False),
    )
    def k(xs_h, ys_h, zs_h, gtab_h, out_h, xv, yv, zv, rowb,
          idxb0, idxb1, rows0, rows1, semg0, semg1, semo0, semo1):
        wid = lax.axis_index("c") * 16 + lax.axis_index("s")
        pid0 = wid * PPW
        b = pid0 // N
        i0 = pid0 - b * N
        pltpu.sync_copy(xs_h.at[b], xv)
        pltpu.sync_copy(ys_h.at[b], yv)
        pltpu.sync_copy(zs_h.at[b], zv)
        r2 = jnp.float32(RADIUS * RADIUS)
        iota = lax.iota(jnp.int32, 16)
        jbase = jnp.full((16,), b * N, jnp.int32)

        def splat_lane(vec, lane):
            return jnp.full((16,), jnp.sum(jnp.where(iota == lane, vec, 0)), vec.dtype)

        ones = jnp.full((16,), 1, jnp.int32)

        def point(i_local, t, idxb):
            lane = jnp.full((16,), i_local % 16, jnp.int32)
            cbase = (i_local // 16) * 16
            xi = splat_lane(xv[pl.ds(cbase, 16)], lane)
            yi = splat_lane(yv[pl.ds(cbase, 16)], lane)
            zi = splat_lane(zv[pl.ds(cbase, 16)], lane)

            def chunk_mask(off):
                dx = xv[pl.ds(off, 16)] - xi
                dy = yv[pl.ds(off, 16)] - yi
                dz = zv[pl.ds(off, 16)] - zi
                sq = dx * dx + dy * dy + dz * dz
                return sq < r2

            def cond(c):
                kg, cnt_s, _ = c
                return jnp.logical_and(cnt_s < NS, kg < NCH // UNR)

            def body(c):
                kg, _, cnt_v = c
                prev = None
                for u in range(UNR):
                    off = kg * (UNR * 16) + u * 16
                    m = chunk_mask(off)
                    pc = plsc.cumsum(m.astype(jnp.int32))
                    if prev is not None:
                        pm, ppc, pjv = prev
                        plsc.store_scatter(rowb, [ppc + cnt_v - ones], pjv, mask=pm)
                        cnt_v = cnt_v + plsc.all_reduce_population_count(pm)
                    prev = (m, pc, iota + jnp.full((16,), off, jnp.int32))
                pm, ppc, pjv = prev
                plsc.store_scatter(rowb, [ppc + cnt_v - ones], pjv, mask=pm)
                cnt_v = cnt_v + plsc.all_reduce_population_count(pm)
                return kg + 1, jnp.max(cnt_v), cnt_v

            _, cnt, _ = lax.while_loop(
                cond, body,
                (jnp.int32(0), jnp.int32(0), jnp.zeros((16,), jnp.int32)))
            cnt32 = jnp.minimum(cnt, NS)
            row0 = rowb[pl.ds(0, 16)]
            first = splat_lane(row0, jnp.zeros((16,), jnp.int32))
            cv = jnp.full((16,), cnt32, jnp.int32)
            v0 = jnp.where(iota < cv, row0, first) + jbase
            v1 = jnp.where(iota + 16 < cv, rowb[pl.ds(16, 16)], first) + jbase
            idxb[pl.ds(t * NS, 16)] = v0
            idxb[pl.ds(t * NS + 16, 16)] = v1

        def scan4(g, idxb):
            ibase = i0 + g * GP
            for t in range(GP):
                point(ibase + t, t, idxb)

        def out_ref_at(g):
            return out_h.at[pl.ds((pid0 + g * GP) * NS, GP * NS)]

        def pair(h, carry):
            g0 = 2 * h
            scan4(g0, idxb0)

            @pl.when(h > 0)
            def _():
                pltpu.make_async_copy(rows0, out_ref_at(g0 - 2), semo0).wait()
                pltpu.make_async_copy(gtab_h.at[idxb1], rows1, semg1).wait()
                pltpu.async_copy(rows1, out_ref_at(g0 - 1), semo1)

            pltpu.async_copy(gtab_h.at[idxb0], rows0, semg0)

            g1 = 2 * h + 1
            scan4(g1, idxb1)

            @pl.when(h > 0)
            def _():
                pltpu.make_async_copy(rows1, out_ref_at(g1 - 2), semo1).wait()

            pltpu.make_async_copy(gtab_h.at[idxb0], rows0, semg0).wait()
            pltpu.async_copy(rows0, out_ref_at(g1 - 1), semo0)
            pltpu.async_copy(gtab_h.at[idxb1], rows1, semg1)
            return carry

        ng = PPW // GP
        lax.fori_loop(0, ng // 2, pair, jnp.int32(0))
        pltpu.make_async_copy(rows0, out_ref_at(ng - 2), semo0).wait()
        pltpu.make_async_copy(gtab_h.at[idxb1], rows1, semg1).wait()
        pltpu.async_copy(rows1, out_ref_at(ng - 1), semo1)
        pltpu.make_async_copy(rows1, out_ref_at(ng - 1), semo1).wait()

    return k(xs, ys, zs, gtab)


def _tc_body(g_ref, m_ref, w0_ref, b0_ref, w1_ref, b1_ref, w2_ref, b2_ref, o_ref):
    P = m_ref.shape[0]
    PN = g_ref.shape[0]
    cols = lax.broadcasted_iota(jnp.int32, (P, DP), 1)
    m = m_ref[...] * jnp.where(cols < 3, 2.0, 1.0)
    g = g_ref[...]
    mexp = jnp.broadcast_to(m[:, None, :], (P, NS, DP)).reshape(PN, DP)
    d = g - mexp
    h = d * d
    h = jnp.maximum(jnp.dot(h, w0_ref[...], preferred_element_type=jnp.float32) + b0_ref[...], 0.0)
    h = jnp.maximum(jnp.dot(h, w1_ref[...], preferred_element_type=jnp.float32) + b1_ref[...], 0.0)
    w = jnp.maximum(jnp.dot(h, w2_ref[...], preferred_element_type=jnp.float32) + b2_ref[...], 0.0)
    gx = d[:, 0:3] + 0.5 * mexp[:, 0:3]
    nd = jnp.concatenate([gx * w, w], axis=1)
    s = jnp.sum(nd.reshape(P, NS, 4), axis=1)
    o_ref[...] = (s[:, 0:3] / s[:, 3:4])[None]


def _tc_mlp(ghat, mtab, w0t, b0r, w1t, b1r, w2c, b2r):
    P = 128
    grid = (BN // P,)
    nb = N // P
    return pl.pallas_call(
        _tc_body,
        grid=grid,
        in_specs=[
            pl.BlockSpec((P * NS, DP), lambda g: (g, 0)),
            pl.BlockSpec((P, DP), lambda g: (g, 0)),
            pl.BlockSpec((DP, 64), lambda g: (0, 0)),
            pl.BlockSpec((1, 64), lambda g: (0, 0)),
            pl.BlockSpec((64, 32), lambda g: (0, 0)),
            pl.BlockSpec((1, 32), lambda g: (0, 0)),
            pl.BlockSpec((32, 1), lambda g: (0, 0)),
            pl.BlockSpec((1, 1), lambda g: (0, 0)),
        ],
        out_specs=pl.BlockSpec((1, P, 3), lambda g: (g // nb, g % nb, 0)),
        out_shape=jax.ShapeDtypeStruct((B, N, 3), jnp.float32),
    )(ghat, mtab, w0t, b0r, w1t, b1r, w2c, b2r)


def kernel(xyz, features, W0, b0, W1, b1, W2, b2):
    featT = jnp.transpose(features, (0, 2, 1))
    pad = jnp.zeros((B, N, DP - 3 - C), jnp.float32)
    gtab = jnp.concatenate([xyz, featT, pad], axis=-1).reshape(BN, DP)
    ghat = _sc_ball_gather(xyz[..., 0], xyz[..., 1], xyz[..., 2], gtab)
    w0t = jnp.zeros((DP, 64), jnp.float32).at[: C + 3].set(W0.T)
    out = _tc_mlp(ghat, gtab, w0t, b0.reshape(1, 64), W1.T, b1.reshape(1, 32),
                  W2.T, b2.reshape(1, 1))
    return jnp.transpose(out, (0, 2, 1))


# speedup vs baseline: 1.0420x; 1.0420x over previous
import functools

import jax
import jax.numpy as jnp
from jax import lax
from jax.experimental import pallas as pl
from jax.experimental.pallas import tpu as pltpu
from jax.experimental.pallas import tpu_sc as plsc

RADIUS = 0.2
NS = 32
B, N, C = 4, 4096, 64
BN = B * N
DP = 80
NW = 32
PPW = BN // NW
GP = 4
NCH = N // 16
UNR = 8


def _sc_ball_gather(xs, ys, zs, gtab):
    mesh = plsc.VectorSubcoreMesh(core_axis_name="c", subcore_axis_name="s")

    @functools.partial(
        pl.kernel,
        out_type=jax.ShapeDtypeStruct((BN * NS, DP), jnp.float32),
        mesh=mesh,
        scratch_types=[
            pltpu.VMEM((N,), jnp.float32),
            pltpu.VMEM((N,), jnp.float32),
            pltpu.VMEM((N,), jnp.float32),
            pltpu.VMEM((320,), jnp.int32),
            pltpu.VMEM((GP * NS,), jnp.int32),
            pltpu.VMEM((GP * NS,), jnp.int32),
            pltpu.VMEM((GP * NS, DP), jnp.float32),
            pltpu.VMEM((GP * NS, DP), jnp.float32),
            pltpu.SemaphoreType.DMA,
            pltpu.SemaphoreType.DMA,
            pltpu.SemaphoreType.DMA,
            pltpu.SemaphoreType.DMA,
        ],
        compiler_params=pltpu.CompilerParams(
            needs_layout_passes=False, use_tc_tiling_on_sc=False),
    )
    def k(xs_h, ys_h, zs_h, gtab_h, out_h, xv, yv, zv, rowb,
          idxb0, idxb1, rows0, rows1, semg0, semg1, semo0, semo1):
        wid = lax.axis_index("c") * 16 + lax.axis_index("s")
        pid0 = wid * PPW
        b = pid0 // N
        i0 = pid0 - b * N
        pltpu.sync_copy(xs_h.at[b], xv)
        pltpu.sync_copy(ys_h.at[b], yv)
        pltpu.sync_copy(zs_h.at[b], zv)
        r2 = jnp.float32(RADIUS * RADIUS)
        iota = lax.iota(jnp.int32, 16)
        jbase = jnp.full((16,), b * N, jnp.int32)

        def splat_lane(vec, lane):
            return jnp.full((16,), jnp.sum(jnp.where(iota == lane, vec, 0)), vec.dtype)

        ones = jnp.full((16,), 1, jnp.int32)

        def point(i_local, t, idxb):
            lane = jnp.full((16,), i_local % 16, jnp.int32)
            cbase = (i_local // 16) * 16
            xi = splat_lane(xv[pl.ds(cbase, 16)], lane)
            yi = splat_lane(yv[pl.ds(cbase, 16)], lane)
            zi = splat_lane(zv[pl.ds(cbase, 16)], lane)

            def chunk_mask(off):
                dx = xv[pl.ds(off, 16)] - xi
                dy = yv[pl.ds(off, 16)] - yi
                dz = zv[pl.ds(off, 16)] - zi
                sq = dx * dx + dy * dy + dz * dz
                return sq < r2

            def cond(c):
                kg, cnt_s, _ = c
                return jnp.logical_and(cnt_s < NS, kg < NCH // UNR)

            def body(c):
                kg, _, cnt_v = c
                prev = None
                for u in range(UNR):
                    off = kg * (UNR * 16) + u * 16
                    m = chunk_mask(off)
                    pc = plsc.cumsum(m.astype(jnp.int32))
                    if prev is not None:
                        pm, ppc, pjv = prev
                        plsc.store_scatter(rowb, [ppc + cnt_v - ones], pjv, mask=pm)
                        cnt_v = cnt_v + plsc.all_reduce_population_count(pm)
                    prev = (m, pc, iota + jnp.full((16,), off, jnp.int32))
                pm, ppc, pjv = prev
                plsc.store_scatter(rowb, [ppc + cnt_v - ones], pjv, mask=pm)
                cnt_v = cnt_v + plsc.all_reduce_population_count(pm)
                return kg + 1, jnp.max(cnt_v), cnt_v

            _, cnt, _ = lax.while_loop(
                cond, body,
                (jnp.int32(0), jnp.int32(0), jnp.zeros((16,), jnp.int32)))
            cnt32 = jnp.minimum(cnt, NS)
            row0 = rowb[pl.ds(0, 16)]
            first = splat_lane(row0, jnp.zeros((16,), jnp.int32))
            cv = jnp.full((16,), cnt32, jnp.int32)
            v0 = jnp.where(iota < cv, row0, first) + jbase
            v1 = jnp.where(iota + 16 < cv, rowb[pl.ds(16, 16)], first) + jbase
            idxb[pl.ds(t * NS, 16)] = v0
            idxb[pl.ds(t * NS + 16, 16)] = v1

        def scan4(g, idxb):
            ibase = i0 + g * GP
            for t in range(GP):
                point(ibase + t, t, idxb)

        def out_ref_at(g):
            return out_h.at[pl.ds((pid0 + g * GP) * NS, GP * NS)]

        def pair(h, carry):
            g0 = 2 * h
            scan4(g0, idxb0)

            @pl.when(h > 0)
            def _():
                pltpu.make_async_copy(rows0, out_ref_at(g0 - 2), semo0).wait()
                pltpu.make_async_copy(gtab_h.at[idxb1], rows1, semg1).wait()
                pltpu.async_copy(rows1, out_ref_at(g0 - 1), semo1)

            pltpu.async_copy(gtab_h.at[idxb0], rows0, semg0)

            g1 = 2 * h + 1
            scan4(g1, idxb1)

            @pl.when(h > 0)
            def _():
                pltpu.make_async_copy(rows1, out_ref_at(g1 - 2), semo1).wait()

            pltpu.make_async_copy(gtab_h.at[idxb0], rows0, semg0).wait()
            pltpu.async_copy(rows0, out_ref_at(g1 - 1), semo0)
            pltpu.async_copy(gtab_h.at[idxb1], rows1, semg1)
            return carry

        ng = PPW // GP
        lax.fori_loop(0, ng // 2, pair, jnp.int32(0))
        pltpu.make_async_copy(rows0, out_ref_at(ng - 2), semo0).wait()
        pltpu.make_async_copy(gtab_h.at[idxb1], rows1, semg1).wait()
        pltpu.async_copy(rows1, out_ref_at(ng - 1), semo1)
        pltpu.make_async_copy(rows1, out_ref_at(ng - 1), semo1).wait()

    return k(xs, ys, zs, gtab)


def _tc_body(g_ref, m_ref, w0_ref, b0_ref, w1_ref, b1_ref, w2_ref, b2_ref, o_ref):
    P = m_ref.shape[0]
    PN = g_ref.shape[0]
    cols = lax.broadcasted_iota(jnp.int32, (P, DP), 1)
    m = m_ref[...] * jnp.where(cols < 3, 2.0, 1.0)
    g = g_ref[...]
    mexp = jnp.broadcast_to(m[:, None, :], (P, NS, DP)).reshape(PN, DP)
    d = g - mexp
    h = d * d
    h = jnp.maximum(jnp.dot(h, w0_ref[...], preferred_element_type=jnp.float32) + b0_ref[...], 0.0)
    h = jnp.maximum(jnp.dot(h, w1_ref[...], preferred_element_type=jnp.float32) + b1_ref[...], 0.0)
    w = jnp.maximum(jnp.dot(h, w2_ref[...], preferred_element_type=jnp.float32) + b2_ref[...], 0.0)
    gx = d[:, 0:3] + 0.5 * mexp[:, 0:3]
    nd = jnp.concatenate([gx * w, w], axis=1)
    s = jnp.sum(nd.reshape(P, NS, 4), axis=1)
    o_ref[...] = (s[:, 0:3] / s[:, 3:4])[None]


def _tc_mlp(ghat, mtab, w0t, b0r, w1t, b1r, w2c, b2r):
    P = 128
    grid = (BN // P,)
    nb = N // P
    return pl.pallas_call(
        _tc_body,
        grid=grid,
        in_specs=[
            pl.BlockSpec((P * NS, DP), lambda g: (g, 0)),
            pl.BlockSpec((P, DP), lambda g: (g, 0)),
            pl.BlockSpec((DP, 64), lambda g: (0, 0)),
            pl.BlockSpec((1, 64), lambda g: (0, 0)),
            pl.BlockSpec((64, 32), lambda g: (0, 0)),
            pl.BlockSpec((1, 32), lambda g: (0, 0)),
            pl.BlockSpec((32, 1), lambda g: (0, 0)),
            pl.BlockSpec((1, 1), lambda g: (0, 0)),
        ],
        out_specs=pl.BlockSpec((1, P, 3), lambda g: (g // nb, g % nb, 0)),
        out_shape=jax.ShapeDtypeStruct((B, N, 3), jnp.float32),
    )(ghat, mtab, w0t, b0r, w1t, b1r, w2c, b2r)


def kernel(xyz, features, W0, b0, W1, b1, W2, b2):
    featT = jnp.transpose(features, (0, 2, 1))
    pad = jnp.zeros((B, N, DP - 3 - C), jnp.float32)
    gtab = jnp.concatenate([xyz, featT, pad], axis=-1).reshape(BN, DP)
    ghat = _sc_ball_gather(xyz[..., 0], xyz[..., 1], xyz[..., 2], gtab)
    w0t = jnp.zeros((DP, 64), jnp.float32).at[: C + 3].set(W0.T)
    out = _tc_mlp(ghat, gtab, w0t, b0.reshape(1, 64), W1.T, b1.reshape(1, 32),
                  W2.T, b2.reshape(1, 1))
    return jnp.transpose(out, (0, 2, 1))


# speedup vs baseline: 1.3044x; 1.2519x over previous
import functools

import jax
import jax.numpy as jnp
from jax import lax
from jax.experimental import pallas as pl
from jax.experimental.pallas import tpu as pltpu
from jax.experimental.pallas import tpu_sc as plsc

RADIUS = 0.2
NS = 32
B, N, C = 4, 4096, 64
BN = B * N
DP = 80
NW = 32
PPW = BN // NW
GP = 4
NCH = N // 16
UNR = 8


def _sc_ball_gather(xs, ys, zs, gtab, pbase, npts):
    ppw = npts // NW
    mesh = plsc.VectorSubcoreMesh(core_axis_name="c", subcore_axis_name="s")

    @functools.partial(
        pl.kernel,
        out_type=jax.ShapeDtypeStruct((npts * NS, DP), jnp.float32),
        mesh=mesh,
        scratch_types=[
            pltpu.VMEM((N,), jnp.float32),
            pltpu.VMEM((N,), jnp.float32),
            pltpu.VMEM((N,), jnp.float32),
            pltpu.VMEM((320,), jnp.int32),
            pltpu.VMEM((GP * NS,), jnp.int32),
            pltpu.VMEM((GP * NS,), jnp.int32),
            pltpu.VMEM((GP * NS, DP), jnp.float32),
            pltpu.VMEM((GP * NS, DP), jnp.float32),
            pltpu.SemaphoreType.DMA,
            pltpu.SemaphoreType.DMA,
            pltpu.SemaphoreType.DMA,
            pltpu.SemaphoreType.DMA,
        ],
        compiler_params=pltpu.CompilerParams(
            needs_layout_passes=False, use_tc_tiling_on_sc=False),
    )
    def k(xs_h, ys_h, zs_h, gtab_h, out_h, xv, yv, zv, rowb,
          idxb0, idxb1, rows0, rows1, semg0, semg1, semo0, semo1):
        wid = lax.axis_index("c") * 16 + lax.axis_index("s")
        pid0 = pbase + wid * ppw
        b = pid0 // N
        i0 = pid0 - b * N
        pltpu.sync_copy(xs_h.at[b], xv)
        pltpu.sync_copy(ys_h.at[b], yv)
        pltpu.sync_copy(zs_h.at[b], zv)
        r2 = jnp.float32(RADIUS * RADIUS)
        iota = lax.iota(jnp.int32, 16)
        jbase = jnp.full((16,), b * N, jnp.int32)

        def splat_lane(vec, lane):
            return jnp.full((16,), jnp.sum(jnp.where(iota == lane, vec, 0)), vec.dtype)

        ones = jnp.full((16,), 1, jnp.int32)

        def point(i_local, t, idxb):
            lane = jnp.full((16,), i_local % 16, jnp.int32)
            cbase = (i_local // 16) * 16
            xi = splat_lane(xv[pl.ds(cbase, 16)], lane)
            yi = splat_lane(yv[pl.ds(cbase, 16)], lane)
            zi = splat_lane(zv[pl.ds(cbase, 16)], lane)

            def chunk_mask(off):
                dx = xv[pl.ds(off, 16)] - xi
                dy = yv[pl.ds(off, 16)] - yi
                dz = zv[pl.ds(off, 16)] - zi
                sq = dx * dx + dy * dy + dz * dz
                return sq < r2

            def cond(c):
                kg, cnt_s, _ = c
                return jnp.logical_and(cnt_s < NS, kg < NCH // UNR)

            def body(c):
                kg, _, cnt_v = c
                prev = None
                for u in range(UNR):
                    off = kg * (UNR * 16) + u * 16
                    m = chunk_mask(off)
                    pc = plsc.cumsum(m.astype(jnp.int32))
                    if prev is not None:
                        pm, ppc, pjv = prev
                        plsc.store_scatter(rowb, [ppc + cnt_v - ones], pjv, mask=pm)
                        cnt_v = cnt_v + plsc.all_reduce_population_count(pm)
                    prev = (m, pc, iota + jnp.full((16,), off, jnp.int32))
                pm, ppc, pjv = prev
                plsc.store_scatter(rowb, [ppc + cnt_v - ones], pjv, mask=pm)
                cnt_v = cnt_v + plsc.all_reduce_population_count(pm)
                return kg + 1, jnp.max(cnt_v), cnt_v

            _, cnt, _ = lax.while_loop(
                cond, body,
                (jnp.int32(0), jnp.int32(0), jnp.zeros((16,), jnp.int32)))
            cnt32 = jnp.minimum(cnt, NS)
            row0 = rowb[pl.ds(0, 16)]
            first = splat_lane(row0, jnp.zeros((16,), jnp.int32))
            cv = jnp.full((16,), cnt32, jnp.int32)
            v0 = jnp.where(iota < cv, row0, first) + jbase
            v1 = jnp.where(iota + 16 < cv, rowb[pl.ds(16, 16)], first) + jbase
            idxb[pl.ds(t * NS, 16)] = v0
            idxb[pl.ds(t * NS + 16, 16)] = v1

        def scan4(g, idxb):
            ibase = i0 + g * GP
            for t in range(GP):
                point(ibase + t, t, idxb)

        def out_ref_at(g):
            return out_h.at[pl.ds((pid0 - pbase + g * GP) * NS, GP * NS)]

        def pair(h, carry):
            g0 = 2 * h
            scan4(g0, idxb0)

            @pl.when(h > 0)
            def _():
                pltpu.make_async_copy(rows0, out_ref_at(g0 - 2), semo0).wait()
                pltpu.make_async_copy(gtab_h.at[idxb1], rows1, semg1).wait()
                pltpu.async_copy(rows1, out_ref_at(g0 - 1), semo1)

            pltpu.async_copy(gtab_h.at[idxb0], rows0, semg0)

            g1 = 2 * h + 1
            scan4(g1, idxb1)

            @pl.when(h > 0)
            def _():
                pltpu.make_async_copy(rows1, out_ref_at(g1 - 2), semo1).wait()

            pltpu.make_async_copy(gtab_h.at[idxb0], rows0, semg0).wait()
            pltpu.async_copy(rows0, out_ref_at(g1 - 1), semo0)
            pltpu.async_copy(gtab_h.at[idxb1], rows1, semg1)
            return carry

        ng = ppw // GP
        lax.fori_loop(0, ng // 2, pair, jnp.int32(0))
        pltpu.make_async_copy(rows0, out_ref_at(ng - 2), semo0).wait()
        pltpu.make_async_copy(gtab_h.at[idxb1], rows1, semg1).wait()
        pltpu.async_copy(rows1, out_ref_at(ng - 1), semo1)
        pltpu.make_async_copy(rows1, out_ref_at(ng - 1), semo1).wait()

    return k(xs, ys, zs, gtab)


def _tc_body(g_ref, m_ref, w0_ref, b0_ref, w1_ref, b1_ref, w2_ref, b2_ref, o_ref):
    P = m_ref.shape[0]
    PN = g_ref.shape[0]
    cols = lax.broadcasted_iota(jnp.int32, (P, DP), 1)
    m = m_ref[...] * jnp.where(cols < 3, 2.0, 1.0)
    g = g_ref[...]
    mexp = jnp.broadcast_to(m[:, None, :], (P, NS, DP)).reshape(PN, DP)
    d = g - mexp
    h = d * d
    h = jnp.maximum(jnp.dot(h, w0_ref[...], preferred_element_type=jnp.float32) + b0_ref[...], 0.0)
    h = jnp.maximum(jnp.dot(h, w1_ref[...], preferred_element_type=jnp.float32) + b1_ref[...], 0.0)
    w = jnp.maximum(jnp.dot(h, w2_ref[...], preferred_element_type=jnp.float32) + b2_ref[...], 0.0)
    gx = d[:, 0:3] + 0.5 * mexp[:, 0:3]
    nd = jnp.concatenate([gx * w, w], axis=1)
    s = jnp.sum(nd.reshape(P, NS, 4), axis=1)
    o_ref[...] = s[:, 0:3] / s[:, 3:4]


def _tc_mlp(ghat, gtab, w0t, b0r, w1t, b1r, w2c, b2r, pbase, npts):
    P = 128
    grid = (npts // P,)
    bb = pbase // P
    return pl.pallas_call(
        _tc_body,
        grid=grid,
        in_specs=[
            pl.BlockSpec((P * NS, DP), lambda g: (g, 0)),
            pl.BlockSpec((P, DP), lambda g: (g + bb, 0)),
            pl.BlockSpec((DP, 64), lambda g: (0, 0)),
            pl.BlockSpec((1, 64), lambda g: (0, 0)),
            pl.BlockSpec((64, 32), lambda g: (0, 0)),
            pl.BlockSpec((1, 32), lambda g: (0, 0)),
            pl.BlockSpec((32, 1), lambda g: (0, 0)),
            pl.BlockSpec((1, 1), lambda g: (0, 0)),
        ],
        out_specs=pl.BlockSpec((P, 3), lambda g: (g, 0)),
        out_shape=jax.ShapeDtypeStruct((npts, 3), jnp.float32),
    )(ghat, gtab, w0t, b0r, w1t, b1r, w2c, b2r)


def kernel(xyz, features, W0, b0, W1, b1, W2, b2):
    featT = jnp.transpose(features, (0, 2, 1))
    pad = jnp.zeros((B, N, DP - 3 - C), jnp.float32)
    gtab = jnp.concatenate([xyz, featT, pad], axis=-1).reshape(BN, DP)
    w0t = jnp.zeros((DP, 64), jnp.float32).at[: C + 3].set(W0.T)
    xs, ys, zs = xyz[..., 0], xyz[..., 1], xyz[..., 2]
    wargs = (w0t, b0.reshape(1, 64), W1.T, b1.reshape(1, 32), W2.T,
             b2.reshape(1, 1))
    halves = []
    half = BN // 2
    for pbase in (0, half):
        ghat = _sc_ball_gather(xs, ys, zs, gtab, pbase, half)
        halves.append(_tc_mlp(ghat, gtab, *wargs, pbase, half))
    out = jnp.concatenate(halves, axis=0).reshape(B, N, 3)
    return jnp.transpose(out, (0, 2, 1))


# speedup vs baseline: 1.4562x; 1.1163x over previous
import functools

import jax
import jax.numpy as jnp
from jax import lax
from jax.experimental import pallas as pl
from jax.experimental.pallas import tpu as pltpu
from jax.experimental.pallas import tpu_sc as plsc

RADIUS = 0.2
NS = 32
B, N, C = 4, 4096, 64
BN = B * N
DP = 80
NW = 32
PPW = BN // NW
GP = 4
NCH = N // 16
UNR = 8


def _sc_ball_gather(xs, ys, zs, gtab, pbase, npts):
    ppw = npts // NW
    mesh = plsc.VectorSubcoreMesh(core_axis_name="c", subcore_axis_name="s")

    @functools.partial(
        pl.kernel,
        out_type=jax.ShapeDtypeStruct((npts * NS, DP), jnp.float32),
        mesh=mesh,
        scratch_types=[
            pltpu.VMEM((N,), jnp.float32),
            pltpu.VMEM((N,), jnp.float32),
            pltpu.VMEM((N,), jnp.float32),
            pltpu.VMEM((320,), jnp.int32),
            pltpu.VMEM((GP * NS,), jnp.int32),
            pltpu.VMEM((GP * NS,), jnp.int32),
            pltpu.VMEM((GP * NS, DP), jnp.float32),
            pltpu.VMEM((GP * NS, DP), jnp.float32),
            pltpu.SemaphoreType.DMA,
            pltpu.SemaphoreType.DMA,
            pltpu.SemaphoreType.DMA,
            pltpu.SemaphoreType.DMA,
        ],
        compiler_params=pltpu.CompilerParams(
            needs_layout_passes=False, use_tc_tiling_on_sc=False),
    )
    def k(xs_h, ys_h, zs_h, gtab_h, out_h, xv, yv, zv, rowb,
          idxb0, idxb1, rows0, rows1, semg0, semg1, semo0, semo1):
        wid = lax.axis_index("c") * 16 + lax.axis_index("s")
        pid0 = pbase + wid * ppw
        b = pid0 // N
        i0 = pid0 - b * N
        pltpu.sync_copy(xs_h.at[b], xv)
        pltpu.sync_copy(ys_h.at[b], yv)
        pltpu.sync_copy(zs_h.at[b], zv)
        r2 = jnp.float32(RADIUS * RADIUS)
        iota = lax.iota(jnp.int32, 16)
        jbase = jnp.full((16,), b * N, jnp.int32)

        def splat_lane(vec, lane):
            return jnp.full((16,), jnp.sum(jnp.where(iota == lane, vec, 0)), vec.dtype)

        ones = jnp.full((16,), 1, jnp.int32)

        def point(i_local, t, idxb):
            lane = jnp.full((16,), i_local % 16, jnp.int32)
            cbase = (i_local // 16) * 16
            xi = splat_lane(xv[pl.ds(cbase, 16)], lane)
            yi = splat_lane(yv[pl.ds(cbase, 16)], lane)
            zi = splat_lane(zv[pl.ds(cbase, 16)], lane)

            def chunk_mask(off):
                dx = xv[pl.ds(off, 16)] - xi
                dy = yv[pl.ds(off, 16)] - yi
                dz = zv[pl.ds(off, 16)] - zi
                sq = dx * dx + dy * dy + dz * dz
                return sq < r2

            def cond(c):
                kg, cnt_s, _ = c
                return jnp.logical_and(cnt_s < NS, kg < NCH // UNR)

            def body(c):
                kg, _, cnt_v = c
                prev = None
                for u in range(UNR):
                    off = kg * (UNR * 16) + u * 16
                    m = chunk_mask(off)
                    pc = plsc.cumsum(m.astype(jnp.int32))
                    if prev is not None:
                        pm, ppc, pjv = prev
                        plsc.store_scatter(rowb, [ppc + cnt_v - ones], pjv, mask=pm)
                        cnt_v = cnt_v + plsc.all_reduce_population_count(pm)
                    prev = (m, pc, iota + jnp.full((16,), off, jnp.int32))
                pm, ppc, pjv = prev
                plsc.store_scatter(rowb, [ppc + cnt_v - ones], pjv, mask=pm)
                cnt_v = cnt_v + plsc.all_reduce_population_count(pm)
                return kg + 1, jnp.max(cnt_v), cnt_v

            _, cnt, _ = lax.while_loop(
                cond, body,
                (jnp.int32(0), jnp.int32(0), jnp.zeros((16,), jnp.int32)))
            cnt32 = jnp.minimum(cnt, NS)
            row0 = rowb[pl.ds(0, 16)]
            first = splat_lane(row0, jnp.zeros((16,), jnp.int32))
            cv = jnp.full((16,), cnt32, jnp.int32)
            v0 = jnp.where(iota < cv, row0, first) + jbase
            v1 = jnp.where(iota + 16 < cv, rowb[pl.ds(16, 16)], first) + jbase
            idxb[pl.ds(t * NS, 16)] = v0
            idxb[pl.ds(t * NS + 16, 16)] = v1

        def scan4(g, idxb):
            ibase = i0 + g * GP
            for t in range(GP):
                point(ibase + t, t, idxb)

        def out_ref_at(g):
            return out_h.at[pl.ds((pid0 - pbase + g * GP) * NS, GP * NS)]

        def pair(h, carry):
            g0 = 2 * h
            scan4(g0, idxb0)

            @pl.when(h > 0)
            def _():
                pltpu.make_async_copy(rows0, out_ref_at(g0 - 2), semo0).wait()
                pltpu.make_async_copy(gtab_h.at[idxb1], rows1, semg1).wait()
                pltpu.async_copy(rows1, out_ref_at(g0 - 1), semo1)

            pltpu.async_copy(gtab_h.at[idxb0], rows0, semg0)

            g1 = 2 * h + 1
            scan4(g1, idxb1)

            @pl.when(h > 0)
            def _():
                pltpu.make_async_copy(rows1, out_ref_at(g1 - 2), semo1).wait()

            pltpu.make_async_copy(gtab_h.at[idxb0], rows0, semg0).wait()
            pltpu.async_copy(rows0, out_ref_at(g1 - 1), semo0)
            pltpu.async_copy(gtab_h.at[idxb1], rows1, semg1)
            return carry

        ng = ppw // GP
        lax.fori_loop(0, ng // 2, pair, jnp.int32(0))
        pltpu.make_async_copy(rows0, out_ref_at(ng - 2), semo0).wait()
        pltpu.make_async_copy(gtab_h.at[idxb1], rows1, semg1).wait()
        pltpu.async_copy(rows1, out_ref_at(ng - 1), semo1)
        pltpu.make_async_copy(rows1, out_ref_at(ng - 1), semo1).wait()

    return k(xs, ys, zs, gtab)


def _tc_body(g_ref, m_ref, w0_ref, b0_ref, w1_ref, b1_ref, w2_ref, b2_ref, o_ref):
    P = m_ref.shape[0]
    PN = g_ref.shape[0]
    cols = lax.broadcasted_iota(jnp.int32, (P, DP), 1)
    m = m_ref[...] * jnp.where(cols < 3, 2.0, 1.0)
    g = g_ref[...]
    mexp = jnp.broadcast_to(m[:, None, :], (P, NS, DP)).reshape(PN, DP)
    d = g - mexp
    h = d * d
    h = jnp.maximum(jnp.dot(h, w0_ref[...], preferred_element_type=jnp.float32) + b0_ref[...], 0.0)
    h = jnp.maximum(jnp.dot(h, w1_ref[...], preferred_element_type=jnp.float32) + b1_ref[...], 0.0)
    w = jnp.maximum(jnp.dot(h, w2_ref[...], preferred_element_type=jnp.float32) + b2_ref[...], 0.0)
    gx = d[:, 0:3] + 0.5 * mexp[:, 0:3]
    nd = jnp.concatenate([gx * w, w], axis=1)
    s = jnp.sum(nd.reshape(P, NS, 4), axis=1)
    o_ref[...] = s[:, 0:3] / s[:, 3:4]


def _tc_mlp(ghat, gtab, w0t, b0r, w1t, b1r, w2c, b2r, pbase, npts):
    P = 128
    grid = (npts // P,)
    bb = pbase // P
    return pl.pallas_call(
        _tc_body,
        grid=grid,
        in_specs=[
            pl.BlockSpec((P * NS, DP), lambda g: (g, 0)),
            pl.BlockSpec((P, DP), lambda g: (g + bb, 0)),
            pl.BlockSpec((DP, 64), lambda g: (0, 0)),
            pl.BlockSpec((1, 64), lambda g: (0, 0)),
            pl.BlockSpec((64, 32), lambda g: (0, 0)),
            pl.BlockSpec((1, 32), lambda g: (0, 0)),
            pl.BlockSpec((32, 1), lambda g: (0, 0)),
            pl.BlockSpec((1, 1), lambda g: (0, 0)),
        ],
        out_specs=pl.BlockSpec((P, 3), lambda g: (g, 0)),
        out_shape=jax.ShapeDtypeStruct((npts, 3), jnp.float32),
    )(ghat, gtab, w0t, b0r, w1t, b1r, w2c, b2r)


def kernel(xyz, features, W0, b0, W1, b1, W2, b2):
    featT = jnp.transpose(features, (0, 2, 1))
    pad = jnp.zeros((B, N, DP - 3 - C), jnp.float32)
    gtab = jnp.concatenate([xyz, featT, pad], axis=-1).reshape(BN, DP)
    w0t = jnp.zeros((DP, 64), jnp.float32).at[: C + 3].set(W0.T)
    xs, ys, zs = xyz[..., 0], xyz[..., 1], xyz[..., 2]
    wargs = (w0t, b0.reshape(1, 64), W1.T, b1.reshape(1, 32), W2.T,
             b2.reshape(1, 1))
    parts = []
    q = BN // 4
    for pbase in (0, q, 2 * q, 3 * q):
        ghat = _sc_ball_gather(xs, ys, zs, gtab, pbase, q)
        parts.append(_tc_mlp(ghat, gtab, *wargs, pbase, q))
    out = jnp.concatenate(parts, axis=0).reshape(B, N, 3)
    return jnp.transpose(out, (0, 2, 1))


# speedup vs baseline: 1.4879x; 1.0218x over previous
import functools

import jax
import jax.numpy as jnp
from jax import lax
from jax.experimental import pallas as pl
from jax.experimental.pallas import tpu as pltpu
from jax.experimental.pallas import tpu_sc as plsc

RADIUS = 0.2
NS = 32
B, N, C = 4, 4096, 64
BN = B * N
DP = 80
NW = 32
PPW = BN // NW
GP = 4
NCH = N // 16
UNR = 8


def _sc_ball_gather(xs, ys, zs, gtab, pbase, npts):
    ppw = npts // NW
    mesh = plsc.VectorSubcoreMesh(core_axis_name="c", subcore_axis_name="s")

    @functools.partial(
        pl.kernel,
        out_type=jax.ShapeDtypeStruct((npts * NS, DP), jnp.float32),
        mesh=mesh,
        scratch_types=[
            pltpu.VMEM((N,), jnp.float32),
            pltpu.VMEM((N,), jnp.float32),
            pltpu.VMEM((N,), jnp.float32),
            pltpu.VMEM((320,), jnp.int32),
            pltpu.VMEM((GP * NS,), jnp.int32),
            pltpu.VMEM((GP * NS,), jnp.int32),
            pltpu.VMEM((GP * NS, DP), jnp.float32),
            pltpu.VMEM((GP * NS, DP), jnp.float32),
            pltpu.SemaphoreType.DMA,
            pltpu.SemaphoreType.DMA,
            pltpu.SemaphoreType.DMA,
            pltpu.SemaphoreType.DMA,
        ],
        compiler_params=pltpu.CompilerParams(
            needs_layout_passes=False, use_tc_tiling_on_sc=False),
    )
    def k(xs_h, ys_h, zs_h, gtab_h, out_h, xv, yv, zv, rowb,
          idxb0, idxb1, rows0, rows1, semg0, semg1, semo0, semo1):
        wid = lax.axis_index("c") * 16 + lax.axis_index("s")
        pid0 = pbase + wid * ppw
        b = pid0 // N
        i0 = pid0 - b * N
        pltpu.sync_copy(xs_h.at[b], xv)
        pltpu.sync_copy(ys_h.at[b], yv)
        pltpu.sync_copy(zs_h.at[b], zv)
        r2 = jnp.float32(RADIUS * RADIUS)
        iota = lax.iota(jnp.int32, 16)
        jbase = jnp.full((16,), b * N, jnp.int32)

        def splat_lane(vec, lane):
            return jnp.full((16,), jnp.sum(jnp.where(iota == lane, vec, 0)), vec.dtype)

        ones = jnp.full((16,), 1, jnp.int32)

        def point(i_local, t, idxb):
            lane = jnp.full((16,), i_local % 16, jnp.int32)
            cbase = (i_local // 16) * 16
            xi = splat_lane(xv[pl.ds(cbase, 16)], lane)
            yi = splat_lane(yv[pl.ds(cbase, 16)], lane)
            zi = splat_lane(zv[pl.ds(cbase, 16)], lane)

            def chunk_mask(off):
                dx = xv[pl.ds(off, 16)] - xi
                dy = yv[pl.ds(off, 16)] - yi
                dz = zv[pl.ds(off, 16)] - zi
                sq = dx * dx + dy * dy + dz * dz
                return sq < r2

            def cond(c):
                kg, cnt_s, _ = c
                return jnp.logical_and(cnt_s < NS, kg < NCH // UNR)

            def body(c):
                kg, _, cnt_v = c
                prev = None
                for u in range(UNR):
                    off = kg * (UNR * 16) + u * 16
                    m = chunk_mask(off)
                    pc = plsc.cumsum(m.astype(jnp.int32))
                    if prev is not None:
                        pm, ppc, pjv = prev
                        plsc.store_scatter(rowb, [ppc + cnt_v - ones], pjv, mask=pm)
                        cnt_v = cnt_v + plsc.all_reduce_population_count(pm)
                    prev = (m, pc, iota + jnp.full((16,), off, jnp.int32))
                pm, ppc, pjv = prev
                plsc.store_scatter(rowb, [ppc + cnt_v - ones], pjv, mask=pm)
                cnt_v = cnt_v + plsc.all_reduce_population_count(pm)
                return kg + 1, jnp.max(cnt_v), cnt_v

            _, cnt, _ = lax.while_loop(
                cond, body,
                (jnp.int32(0), jnp.int32(0), jnp.zeros((16,), jnp.int32)))
            cnt32 = jnp.minimum(cnt, NS)
            row0 = rowb[pl.ds(0, 16)]
            first = splat_lane(row0, jnp.zeros((16,), jnp.int32))
            cv = jnp.full((16,), cnt32, jnp.int32)
            v0 = jnp.where(iota < cv, row0, first) + jbase
            v1 = jnp.where(iota + 16 < cv, rowb[pl.ds(16, 16)], first) + jbase
            idxb[pl.ds(t * NS, 16)] = v0
            idxb[pl.ds(t * NS + 16, 16)] = v1

        def scan4(g, idxb):
            ibase = i0 + g * GP
            for t in range(GP):
                point(ibase + t, t, idxb)

        def out_ref_at(g):
            return out_h.at[pl.ds((pid0 - pbase + g * GP) * NS, GP * NS)]

        def pair(h, carry):
            g0 = 2 * h
            scan4(g0, idxb0)

            @pl.when(h > 0)
            def _():
                pltpu.make_async_copy(rows0, out_ref_at(g0 - 2), semo0).wait()
                pltpu.make_async_copy(gtab_h.at[idxb1], rows1, semg1).wait()
                pltpu.async_copy(rows1, out_ref_at(g0 - 1), semo1)

            pltpu.async_copy(gtab_h.at[idxb0], rows0, semg0)

            g1 = 2 * h + 1
            scan4(g1, idxb1)

            @pl.when(h > 0)
            def _():
                pltpu.make_async_copy(rows1, out_ref_at(g1 - 2), semo1).wait()

            pltpu.make_async_copy(gtab_h.at[idxb0], rows0, semg0).wait()
            pltpu.async_copy(rows0, out_ref_at(g1 - 1), semo0)
            pltpu.async_copy(gtab_h.at[idxb1], rows1, semg1)
            return carry

        ng = ppw // GP
        lax.fori_loop(0, ng // 2, pair, jnp.int32(0))
        pltpu.make_async_copy(rows0, out_ref_at(ng - 2), semo0).wait()
        pltpu.make_async_copy(gtab_h.at[idxb1], rows1, semg1).wait()
        pltpu.async_copy(rows1, out_ref_at(ng - 1), semo1)
        pltpu.make_async_copy(rows1, out_ref_at(ng - 1), semo1).wait()

    return k(xs, ys, zs, gtab)


def _tc_body(g_ref, m_ref, w0_ref, b0_ref, w1_ref, b1_ref, w2_ref, b2_ref, o_ref):
    P = m_ref.shape[0]
    PN = g_ref.shape[0]
    cols = lax.broadcasted_iota(jnp.int32, (P, DP), 1)
    m = m_ref[...] * jnp.where(cols < 3, 2.0, 1.0)
    g = g_ref[...]
    mexp = jnp.broadcast_to(m[:, None, :], (P, NS, DP)).reshape(PN, DP)
    d = g - mexp
    h = d * d
    h = jnp.maximum(jnp.dot(h, w0_ref[...], preferred_element_type=jnp.float32) + b0_ref[...], 0.0)
    h = jnp.maximum(jnp.dot(h, w1_ref[...], preferred_element_type=jnp.float32) + b1_ref[...], 0.0)
    w = jnp.maximum(jnp.dot(h, w2_ref[...], preferred_element_type=jnp.float32) + b2_ref[...], 0.0)
    gx = d[:, 0:3] + 0.5 * mexp[:, 0:3]
    nd = jnp.concatenate([gx * w, w], axis=1)
    s = jnp.sum(nd.reshape(P, NS, 4), axis=1)
    o_ref[...] = s[:, 0:3] / s[:, 3:4]


def _tc_mlp(ghat, gtab, w0t, b0r, w1t, b1r, w2c, b2r, pbase, npts):
    P = 128
    grid = (npts // P,)
    bb = pbase // P
    return pl.pallas_call(
        _tc_body,
        grid=grid,
        in_specs=[
            pl.BlockSpec((P * NS, DP), lambda g: (g, 0)),
            pl.BlockSpec((P, DP), lambda g: (g + bb, 0)),
            pl.BlockSpec((DP, 64), lambda g: (0, 0)),
            pl.BlockSpec((1, 64), lambda g: (0, 0)),
            pl.BlockSpec((64, 32), lambda g: (0, 0)),
            pl.BlockSpec((1, 32), lambda g: (0, 0)),
            pl.BlockSpec((32, 1), lambda g: (0, 0)),
            pl.BlockSpec((1, 1), lambda g: (0, 0)),
        ],
        out_specs=pl.BlockSpec((P, 3), lambda g: (g, 0)),
        out_shape=jax.ShapeDtypeStruct((npts, 3), jnp.float32),
    )(ghat, gtab, w0t, b0r, w1t, b1r, w2c, b2r)


def kernel(xyz, features, W0, b0, W1, b1, W2, b2):
    featT = jnp.transpose(features, (0, 2, 1))
    pad = jnp.zeros((B, N, DP - 3 - C), jnp.float32)
    gtab = jnp.concatenate([xyz, featT, pad], axis=-1).reshape(BN, DP)
    w0t = jnp.zeros((DP, 64), jnp.float32).at[: C + 3].set(W0.T)
    xs, ys, zs = xyz[..., 0], xyz[..., 1], xyz[..., 2]
    wargs = (w0t, b0.reshape(1, 64), W1.T, b1.reshape(1, 32), W2.T,
             b2.reshape(1, 1))
    parts = []
    q = BN // 8
    for pbase in range(0, BN, q):
        ghat = _sc_ball_gather(xs, ys, zs, gtab, pbase, q)
        parts.append(_tc_mlp(ghat, gtab, *wargs, pbase, q))
    out = jnp.concatenate(parts, axis=0).reshape(B, N, 3)
    return jnp.transpose(out, (0, 2, 1))


# speedup vs baseline: 1.4889x; 1.0007x over previous
import functools

import jax
import jax.numpy as jnp
from jax import lax
from jax.experimental import pallas as pl
from jax.experimental.pallas import tpu as pltpu
from jax.experimental.pallas import tpu_sc as plsc

RADIUS = 0.2
NS = 32
B, N, C = 4, 4096, 64
BN = B * N
DP = 80
NW = 32
GP = 4
NCH = N // 16
UNR = 8


def _sc_ball_gather(xs, ys, zs, gtab, pbase, npts):
    ppw = npts // NW
    mesh = plsc.VectorSubcoreMesh(core_axis_name="c", subcore_axis_name="s")

    @functools.partial(
        pl.kernel,
        out_type=jax.ShapeDtypeStruct((npts * NS, DP), jnp.float32),
        mesh=mesh,
        scratch_types=[
            pltpu.VMEM((N,), jnp.float32),
            pltpu.VMEM((N,), jnp.float32),
            pltpu.VMEM((N,), jnp.float32),
            pltpu.VMEM((320,), jnp.int32),
            pltpu.VMEM((GP * NS,), jnp.int32),
            pltpu.VMEM((GP * NS,), jnp.int32),
            pltpu.VMEM((GP * NS, DP), jnp.float32),
            pltpu.VMEM((GP * NS, DP), jnp.float32),
            pltpu.SemaphoreType.DMA,
            pltpu.SemaphoreType.DMA,
            pltpu.SemaphoreType.DMA,
            pltpu.SemaphoreType.DMA,
        ],
        compiler_params=pltpu.CompilerParams(
            needs_layout_passes=False, use_tc_tiling_on_sc=False),
    )
    def k(xs_h, ys_h, zs_h, gtab_h, out_h, xv, yv, zv, rowb,
          idxb0, idxb1, rows0, rows1, semg0, semg1, semo0, semo1):
        wid = lax.axis_index("c") * 16 + lax.axis_index("s")
        pid0 = pbase + wid * ppw
        b = pid0 // N
        i0 = pid0 - b * N
        pltpu.sync_copy(xs_h.at[b], xv)
        pltpu.sync_copy(ys_h.at[b], yv)
        pltpu.sync_copy(zs_h.at[b], zv)
        r2 = jnp.float32(RADIUS * RADIUS)
        iota = lax.iota(jnp.int32, 16)
        jbase = jnp.full((16,), b * N, jnp.int32)

        def splat_lane(vec, lane):
            return jnp.full((16,), jnp.sum(jnp.where(iota == lane, vec, 0)), vec.dtype)

        ones = jnp.full((16,), 1, jnp.int32)

        def point(i_local, t, idxb):
            lane = jnp.full((16,), i_local % 16, jnp.int32)
            cbase = (i_local // 16) * 16
            xi = splat_lane(xv[pl.ds(cbase, 16)], lane)
            yi = splat_lane(yv[pl.ds(cbase, 16)], lane)
            zi = splat_lane(zv[pl.ds(cbase, 16)], lane)

            def chunk_mask(off):
                dx = xv[pl.ds(off, 16)] - xi
                dy = yv[pl.ds(off, 16)] - yi
                dz = zv[pl.ds(off, 16)] - zi
                sq = dx * dx + dy * dy + dz * dz
                return sq < r2

            def cond(c):
                kg, cnt_s, _ = c
                return jnp.logical_and(cnt_s < NS, kg < NCH // UNR)

            def body(c):
                kg, _, cnt_v = c
                prev = None
                for u in range(UNR):
                    off = kg * (UNR * 16) + u * 16
                    m = chunk_mask(off)
                    pc = plsc.cumsum(m.astype(jnp.int32))
                    if prev is not None:
                        pm, ppc, pjv = prev
                        plsc.store_scatter(rowb, [ppc + cnt_v - ones], pjv, mask=pm)
                        cnt_v = cnt_v + plsc.all_reduce_population_count(pm)
                    prev = (m, pc, iota + jnp.full((16,), off, jnp.int32))
                pm, ppc, pjv = prev
                plsc.store_scatter(rowb, [ppc + cnt_v - ones], pjv, mask=pm)
                cnt_v = cnt_v + plsc.all_reduce_population_count(pm)
                return kg + 1, jnp.max(cnt_v), cnt_v

            _, cnt, _ = lax.while_loop(
                cond, body,
                (jnp.int32(0), jnp.int32(0), jnp.zeros((16,), jnp.int32)))
            cnt32 = jnp.minimum(cnt, NS)
            row0 = rowb[pl.ds(0, 16)]
            first = splat_lane(row0, jnp.zeros((16,), jnp.int32))
            cv = jnp.full((16,), cnt32, jnp.int32)
            v0 = jnp.where(iota < cv, row0, first) + jbase
            v1 = jnp.where(iota + 16 < cv, rowb[pl.ds(16, 16)], first) + jbase
            idxb[pl.ds(t * NS, 16)] = v0
            idxb[pl.ds(t * NS + 16, 16)] = v1

        def scan4(g, idxb):
            ibase = i0 + g * GP
            for t in range(GP):
                point(ibase + t, t, idxb)

        def out_ref_at(g):
            return out_h.at[pl.ds((pid0 - pbase + g * GP) * NS, GP * NS)]

        def pair(h, carry):
            g0 = 2 * h
            scan4(g0, idxb0)

            @pl.when(h > 0)
            def _():
                pltpu.make_async_copy(rows0, out_ref_at(g0 - 2), semo0).wait()
                pltpu.make_async_copy(gtab_h.at[idxb1], rows1, semg1).wait()
                pltpu.async_copy(rows1, out_ref_at(g0 - 1), semo1)

            pltpu.async_copy(gtab_h.at[idxb0], rows0, semg0)

            g1 = 2 * h + 1
            scan4(g1, idxb1)

            @pl.when(h > 0)
            def _():
                pltpu.make_async_copy(rows1, out_ref_at(g1 - 2), semo1).wait()

            pltpu.make_async_copy(gtab_h.at[idxb0], rows0, semg0).wait()
            pltpu.async_copy(rows0, out_ref_at(g1 - 1), semo0)
            pltpu.async_copy(gtab_h.at[idxb1], rows1, semg1)
            return carry

        ng = ppw // GP
        lax.fori_loop(0, ng // 2, pair, jnp.int32(0))
        pltpu.make_async_copy(rows0, out_ref_at(ng - 2), semo0).wait()
        pltpu.make_async_copy(gtab_h.at[idxb1], rows1, semg1).wait()
        pltpu.async_copy(rows1, out_ref_at(ng - 1), semo1)
        pltpu.make_async_copy(rows1, out_ref_at(ng - 1), semo1).wait()

    return k(xs, ys, zs, gtab)


def _tc_body(g_ref, m_ref, w0_ref, b0_ref, w1_ref, b1_ref, w2_ref, b2_ref, o_ref):
    P = m_ref.shape[0]
    PN = g_ref.shape[0]
    cols = lax.broadcasted_iota(jnp.int32, (P, DP), 1)
    m = m_ref[...] * jnp.where(cols < 3, 2.0, 1.0)
    g = g_ref[...]
    mexp = jnp.broadcast_to(m[:, None, :], (P, NS, DP)).reshape(PN, DP)
    d = g - mexp
    h = d * d
    h = jnp.maximum(jnp.dot(h, w0_ref[...], preferred_element_type=jnp.float32) + b0_ref[...], 0.0)
    h = jnp.maximum(jnp.dot(h, w1_ref[...], preferred_element_type=jnp.float32) + b1_ref[...], 0.0)
    w = jnp.maximum(jnp.dot(h, w2_ref[...], preferred_element_type=jnp.float32) + b2_ref[...], 0.0)
    gx = d[:, 0:3] + 0.5 * mexp[:, 0:3]
    nd = jnp.concatenate([gx * w, w], axis=1)
    s = jnp.sum(nd.reshape(P, NS, 4), axis=1)
    o_ref[...] = s[:, 0:3] / s[:, 3:4]


def _tc_mlp(ghat, gtab, w0t, b0r, w1t, b1r, w2c, b2r, pbase, npts):
    P = 128
    grid = (npts // P,)
    bb = pbase // P
    return pl.pallas_call(
        _tc_body,
        grid=grid,
        in_specs=[
            pl.BlockSpec((P * NS, DP), lambda g: (g, 0)),
            pl.BlockSpec((P, DP), lambda g: (g + bb, 0)),
            pl.BlockSpec((DP, 64), lambda g: (0, 0)),
            pl.BlockSpec((1, 64), lambda g: (0, 0)),
            pl.BlockSpec((64, 32), lambda g: (0, 0)),
            pl.BlockSpec((1, 32), lambda g: (0, 0)),
            pl.BlockSpec((32, 1), lambda g: (0, 0)),
            pl.BlockSpec((1, 1), lambda g: (0, 0)),
        ],
        out_specs=pl.BlockSpec((P, 3), lambda g: (g, 0)),
        out_shape=jax.ShapeDtypeStruct((npts, 3), jnp.float32),
    )(ghat, gtab, w0t, b0r, w1t, b1r, w2c, b2r)


def kernel(xyz, features, W0, b0, W1, b1, W2, b2):
    featT = jnp.transpose(features, (0, 2, 1))
    pad = jnp.zeros((B, N, DP - 3 - C), jnp.float32)
    gtab = jnp.concatenate([xyz, featT, pad], axis=-1).reshape(BN, DP)
    w0t = jnp.zeros((DP, 64), jnp.float32).at[: C + 3].set(W0.T)
    xs, ys, zs = xyz[..., 0], xyz[..., 1], xyz[..., 2]
    wargs = (w0t, b0.reshape(1, 64), W1.T, b1.reshape(1, 32), W2.T,
             b2.reshape(1, 1))
    parts = []
    q = BN // 8
    for pbase in range(0, BN, q):
        ghat = _sc_ball_gather(xs, ys, zs, gtab, pbase, q)
        parts.append(_tc_mlp(ghat, gtab, *wargs, pbase, q))
    out = jnp.concatenate(parts, axis=0).reshape(B, N, 3)
    return jnp.transpose(out, (0, 2, 1))
